# baseline (speedup 1.0000x reference)
import functools

import jax
import jax.numpy as jnp
import numpy as np
from jax.experimental import pallas as pl
from jax.experimental.pallas import tpu as pltpu

_N = 100
_INF = 1000000000.0
_MU = 1e-08
_SAFE = 0.05
_P = 8
_RADIUS = (2.0, 5.0, 10.0)
_BS = 8


def _tc_body(maps_ref, paras_ref, traj_ref, cur_ref, out_ref):
    maps = maps_ref[...]
    paras = paras_ref[...]
    cur = cur_ref[...]
    traj = traj_ref[...]

    o0x = traj[:, 0, 0] + cur[:, 0]
    o0y = traj[:, 0, 1] + cur[:, 1]
    o1x = traj[:, -1, 0] + cur[:, 0]
    o1y = traj[:, -1, 1] + cur[:, 1]
    mvx = o1x - o0x
    mvy = o1y - o0y
    ml = jnp.sqrt(mvx * mvx + mvy * mvy)

    xs = jax.lax.broadcasted_iota(jnp.int32, (_N, _N), 0)[None].astype(jnp.float32)
    ys = jax.lax.broadcasted_iota(jnp.int32, (_N, _N), 1)[None].astype(jnp.float32)
    wx = paras[:, 0][:, None, None]
    wy = paras[:, 1][:, None, None]
    bx = paras[:, 2][:, None, None]
    by = paras[:, 3][:, None, None]
    cx = cur[:, 0][:, None, None]
    cy = cur[:, 1][:, None, None]
    dx = (xs - bx) / wx - cx
    dy = (ys - by) / wy - cy
    d2 = dx * dx + dy * dy
    dist = jnp.sqrt(d2)
    ang = jnp.arctan2(dx, dy)
    q = (ang % (2 * np.pi)) / (2 * np.pi / _P)
    ai = q.astype(jnp.int32)
    unsafe = maps > _SAFE
    v = (dist + _MU) / (maps + _MU)

    mins = []
    for rt in _RADIUS:
        r = (rt * ml)[:, None, None]
        rm = dist <= r
        base = jnp.where(unsafe & rm, v, _INF)
        for a in range(_P):
            d = jnp.where(ai == a, base, _INF)
            md = jnp.min(jnp.min(d, axis=-1), axis=-1)
            mins.append(jnp.where(md < _INF, md, 0.0))
    out_ref[...] = jnp.stack(mins, axis=-1)


def kernel(seg_maps, seg_map_paras, trajectories, current_pos):
    B = seg_maps.shape[0]
    cur = current_pos[:, 0, :]
    grid = (B // _BS,)
    out = pl.pallas_call(
        _tc_body,
        grid=grid,
        in_specs=[
            pl.BlockSpec((_BS, _N, _N), lambda i: (i, 0, 0)),
            pl.BlockSpec((_BS, 4), lambda i: (i, 0)),
            pl.BlockSpec((_BS, 20, 2), lambda i: (i, 0, 0)),
            pl.BlockSpec((_BS, 2), lambda i: (i, 0)),
        ],
        out_specs=pl.BlockSpec((_BS, 24), lambda i: (i, 0)),
        out_shape=jax.ShapeDtypeStruct((B, 24), jnp.float32),
        compiler_params=pltpu.CompilerParams(
            dimension_semantics=("parallel",),
        ),
    )(seg_maps, seg_map_paras, trajectories, cur)
    return jnp.swapaxes(out.reshape(B, 3, _P), -2, -1)
